# speedup vs baseline: 21.2323x; 21.2323x over previous
import jax
import jax.numpy as jnp
from jax.experimental import pallas as pl


B_BLK = 128


def _body(idx_ref, w1_ref, w2_ref, out_ref):
    sel = idx_ref[...][:, :, None]
    w1 = w1_ref[...][None]
    w2 = w2_ref[...][None]
    out_ref[...] = jnp.where(sel == 1, w1, 0.0) + jnp.where(sel == 2, w2, 0.0)


def kernel(inputs, w1, w2):
    B, T = inputs.shape
    U = w1.shape[1]
    idx = inputs.astype(jnp.int32)
    grid = (B // B_BLK,)
    return pl.pallas_call(
        _body,
        grid=grid,
        in_specs=[
            pl.BlockSpec((B_BLK, T), lambda i: (i, 0)),
            pl.BlockSpec((1, U), lambda i: (0, 0)),
            pl.BlockSpec((1, U), lambda i: (0, 0)),
        ],
        out_specs=pl.BlockSpec((B_BLK, T, U), lambda i: (i, 0, 0)),
        out_shape=jax.ShapeDtypeStruct((B, T, U), jnp.float32),
    )(idx, w1, w2)
